# baseline (speedup 1.0000x reference)
import functools

import jax
import jax.numpy as jnp
from jax import lax
from jax.experimental import pallas as pl
from jax.experimental.pallas import tpu as pltpu
from jax.experimental.pallas import tpu_sc as plsc

_ROW = 128
_TABLE_PAD = 1024


@functools.lru_cache(maxsize=None)
def _make_lookup(n_flat: int, num_cores: int, num_subcores: int):
    num_workers = num_cores * num_subcores
    chunk = n_flat // num_workers
    n_gathers = chunk // _ROW
    assert chunk * num_workers == n_flat and n_gathers * _ROW == chunk
    mesh = plsc.VectorSubcoreMesh(
        core_axis_name="c", subcore_axis_name="s", num_cores=num_cores)

    @functools.partial(
        pl.kernel,
        mesh=mesh,
        out_type=jax.ShapeDtypeStruct((n_flat,), jnp.float32),
        scratch_types=[
            pltpu.VMEM_SHARED((_TABLE_PAD,), jnp.float32),
            pltpu.VMEM((chunk,), jnp.int32),
            pltpu.VMEM((chunk,), jnp.float32),
            pltpu.SemaphoreType.DMA,
        ],
    )
    def lookup(ents_hbm, table_hbm, out_hbm, table_sh, idx_v, out_v, sem):
        cid = lax.axis_index("c")
        sid = lax.axis_index("s")
        wid = sid * num_cores + cid
        base = wid * chunk

        @pl.when(sid == 0)
        def _stage_table():
            pltpu.sync_copy(table_hbm, table_sh)

        pltpu.sync_copy(ents_hbm.at[pl.ds(base, chunk)], idx_v)
        plsc.subcore_barrier()
        copies = [
            pltpu.async_copy(table_sh.at[idx_v.at[pl.ds(j * _ROW, _ROW)]],
                             out_v.at[pl.ds(j * _ROW, _ROW)], sem)
            for j in range(n_gathers)
        ]
        for c in copies:
            c.wait()
        pltpu.sync_copy(out_v, out_hbm.at[pl.ds(base, chunk)])

    return lookup


def kernel(entities, ent2cluster_key, ent2cluster_value):
    del ent2cluster_key
    shape = entities.shape
    n = entities.size
    flat = entities.reshape(-1).astype(jnp.int32)
    table = jnp.zeros((_TABLE_PAD,), jnp.float32).at[:ent2cluster_value.shape[0]].set(
        ent2cluster_value.astype(jnp.float32))
    info = plsc.get_sparse_core_info()
    out = _make_lookup(n, info.num_cores, info.num_subcores)(flat, table)
    return out.reshape(shape)


# speedup vs baseline: 1.0635x; 1.0635x over previous
import functools

import jax
import jax.numpy as jnp
from jax import lax
from jax.experimental import pallas as pl
from jax.experimental.pallas import tpu as pltpu
from jax.experimental.pallas import tpu_sc as plsc

_ROW = 128
_TABLE_PAD = 1024


@functools.lru_cache(maxsize=None)
def _make_lookup(n_flat: int, num_cores: int, num_subcores: int):
    num_workers = num_cores * num_subcores
    chunk = n_flat // num_workers
    n_gathers = chunk // _ROW
    assert chunk * num_workers == n_flat and n_gathers * _ROW == chunk
    mesh = plsc.VectorSubcoreMesh(
        core_axis_name="c", subcore_axis_name="s", num_cores=num_cores)

    @functools.partial(
        pl.kernel,
        mesh=mesh,
        out_type=jax.ShapeDtypeStruct((n_flat,), jnp.float32),
        scratch_types=[
            pltpu.VMEM_SHARED((_TABLE_PAD,), jnp.float32),
            pltpu.VMEM((chunk,), jnp.int32),
            pltpu.VMEM((chunk,), jnp.float32),
            pltpu.SemaphoreType.DMA,
        ],
    )
    def lookup(ents_hbm, table_hbm, out_hbm, table_sh, idx_v, out_v, sem):
        cid = lax.axis_index("c")
        sid = lax.axis_index("s")
        wid = sid * num_cores + cid
        base = wid * chunk

        @pl.when(sid == 0)
        def _stage_table():
            pltpu.sync_copy(table_hbm, table_sh)

        pltpu.sync_copy(ents_hbm.at[pl.ds(base, chunk)], idx_v)
        plsc.subcore_barrier()
        copies = [
            pltpu.async_copy(table_sh.at[idx_v.at[pl.ds(j * _ROW, _ROW)]],
                             out_v.at[pl.ds(j * _ROW, _ROW)], sem)
            for j in range(n_gathers)
        ]
        for c in copies:
            c.wait()
        pltpu.sync_copy(out_v, out_hbm.at[pl.ds(base, chunk)])

    return lookup


def kernel(entities, ent2cluster_key, ent2cluster_value):
    del ent2cluster_key
    shape = entities.shape
    n = entities.size
    flat = entities.reshape(-1).astype(jnp.int32)
    table = jnp.zeros((_TABLE_PAD,), jnp.float32).at[:ent2cluster_value.shape[0]].set(
        ent2cluster_value.astype(jnp.float32))
    info = plsc.get_sparse_core_info()
    out = _make_lookup(n, 1, info.num_subcores)(flat, table)
    return out.reshape(shape)


# speedup vs baseline: 1.0662x; 1.0026x over previous
import functools

import jax
import jax.numpy as jnp
from jax import lax
from jax.experimental import pallas as pl
from jax.experimental.pallas import tpu as pltpu
from jax.experimental.pallas import tpu_sc as plsc

_ROW = 128
_TABLE_PAD = 1024


@functools.lru_cache(maxsize=None)
def _make_lookup(n_flat: int, table_n: int, num_cores: int, num_subcores: int):
    num_workers = num_cores * num_subcores
    chunk = n_flat // num_workers
    n_gathers = chunk // _ROW
    assert chunk * num_workers == n_flat and n_gathers * _ROW == chunk
    mesh = plsc.VectorSubcoreMesh(
        core_axis_name="c", subcore_axis_name="s", num_cores=num_cores)

    @functools.partial(
        pl.kernel,
        mesh=mesh,
        out_type=jax.ShapeDtypeStruct((n_flat,), jnp.float32),
        scratch_types=[
            pltpu.VMEM_SHARED((table_n,), jnp.float32),
            pltpu.VMEM((chunk,), jnp.int32),
            pltpu.VMEM((chunk,), jnp.float32),
            pltpu.SemaphoreType.DMA,
        ],
    )
    def lookup(ents_hbm, table_hbm, out_hbm, table_sh, idx_v, out_v, sem):
        cid = lax.axis_index("c")
        sid = lax.axis_index("s")
        wid = sid * num_cores + cid
        base = wid * chunk

        @pl.when(sid == 0)
        def _stage_table():
            pltpu.sync_copy(table_hbm, table_sh)

        pltpu.sync_copy(ents_hbm.at[pl.ds(base, chunk)], idx_v)
        plsc.subcore_barrier()
        copies = [
            pltpu.async_copy(table_sh.at[idx_v.at[pl.ds(j * _ROW, _ROW)]],
                             out_v.at[pl.ds(j * _ROW, _ROW)], sem)
            for j in range(n_gathers)
        ]
        for c in copies:
            c.wait()
        pltpu.sync_copy(out_v, out_hbm.at[pl.ds(base, chunk)])

    return lookup


def kernel(entities, ent2cluster_key, ent2cluster_value):
    del ent2cluster_key
    shape = entities.shape
    n = entities.size
    flat = entities.reshape(-1).astype(jnp.int32)
    table = ent2cluster_value.astype(jnp.float32)
    info = plsc.get_sparse_core_info()
    out = _make_lookup(n, table.shape[0], 1, info.num_subcores)(flat, table)
    return out.reshape(shape)


# speedup vs baseline: 1.0691x; 1.0027x over previous
import functools

import jax
import jax.numpy as jnp
from jax import lax
from jax.experimental import pallas as pl
from jax.experimental.pallas import tpu as pltpu
from jax.experimental.pallas import tpu_sc as plsc

_ROW = 128
_TABLE_PAD = 1024


@functools.lru_cache(maxsize=None)
def _make_lookup(n_flat: int, table_n: int, num_cores: int, num_subcores: int):
    num_workers = num_cores * num_subcores
    chunk = n_flat // num_workers
    n_gathers = chunk // _ROW
    assert chunk * num_workers == n_flat and n_gathers * _ROW == chunk
    mesh = plsc.VectorSubcoreMesh(
        core_axis_name="c", subcore_axis_name="s", num_cores=num_cores)

    @functools.partial(
        pl.kernel,
        mesh=mesh,
        out_type=jax.ShapeDtypeStruct((n_flat,), jnp.float32),
        scratch_types=[
            pltpu.VMEM_SHARED((table_n,), jnp.float32),
            pltpu.VMEM((chunk,), jnp.int32),
            pltpu.VMEM((chunk,), jnp.float32),
            pltpu.SemaphoreType.DMA,
        ],
    )
    def lookup(ents_hbm, table_hbm, out_hbm, table_sh, idx_v, out_v, sem):
        cid = lax.axis_index("c")
        sid = lax.axis_index("s")
        wid = sid * num_cores + cid
        base = wid * chunk

        @pl.when(sid == 0)
        def _stage_table():
            pltpu.sync_copy(table_hbm, table_sh)

        pltpu.sync_copy(ents_hbm.at[pl.ds(base, chunk)], idx_v)
        plsc.subcore_barrier()
        pltpu.async_copy(table_sh.at[idx_v], out_v, sem).wait()
        pltpu.sync_copy(out_v, out_hbm.at[pl.ds(base, chunk)])

    return lookup


def kernel(entities, ent2cluster_key, ent2cluster_value):
    del ent2cluster_key
    shape = entities.shape
    n = entities.size
    flat = entities.reshape(-1).astype(jnp.int32)
    table = ent2cluster_value.astype(jnp.float32)
    info = plsc.get_sparse_core_info()
    out = _make_lookup(n, table.shape[0], 1, info.num_subcores)(flat, table)
    return out.reshape(shape)
